# baseline (speedup 1.0000x reference)
import functools

import jax
import jax.numpy as jnp
from jax import lax
from jax.experimental import pallas as pl
from jax.experimental.pallas import tpu as pltpu
from jax.experimental.pallas import tpu_sc as plsc

_N_ROWS = 1_000_000


def kernel(item_emb):
    info = plsc.get_sparse_core_info()
    nc, ns = info.num_cores, info.num_subcores
    rows_per_tile = _N_ROWS // (nc * ns)
    mesh = plsc.VectorSubcoreMesh(core_axis_name="c", subcore_axis_name="s")

    @functools.partial(
        pl.kernel,
        mesh=mesh,
        out_type=jax.ShapeDtypeStruct(item_emb.shape, item_emb.dtype),
    )
    def copy_kernel(in_hbm, out_hbm):
        wid = lax.axis_index("s") * nc + lax.axis_index("c")
        base = pl.multiple_of(lax.div(wid * rows_per_tile, 8) * 8, 8)
        sz = rows_per_tile + 8 - rows_per_tile % 8
        pltpu.sync_copy(
            in_hbm.at[pl.ds(base, sz)],
            out_hbm.at[pl.ds(base, sz)],
        )

    return copy_kernel(item_emb)


# speedup vs baseline: 16.8543x; 16.8543x over previous
import functools

import jax
import jax.numpy as jnp
from jax import lax
from jax.experimental import pallas as pl
from jax.experimental.pallas import tpu as pltpu
from jax.experimental.pallas import tpu_sc as plsc

_N_ROWS = 1_000_000
_CH = 480


def kernel(item_emb):
    info = plsc.get_sparse_core_info()
    nc, ns = info.num_cores, info.num_subcores
    rows_per_tile = _N_ROWS // (nc * ns)
    span = rows_per_tile + 8 - rows_per_tile % 8
    n_full = span // _CH
    offs = [k * _CH for k in range(n_full)]
    if n_full * _CH < span:
        offs.append(span - _CH)
    n = len(offs)

    mesh = plsc.VectorSubcoreMesh(core_axis_name="c", subcore_axis_name="s")

    @functools.partial(
        pl.kernel,
        mesh=mesh,
        out_type=jax.ShapeDtypeStruct(item_emb.shape, item_emb.dtype),
        scratch_types=[
            pltpu.VMEM((2, _CH, 32), jnp.float32),
            pltpu.SemaphoreType.DMA((2,)),
            pltpu.SemaphoreType.DMA((2,)),
        ],
    )
    def copy_kernel(in_hbm, out_hbm, bufs, rsem, wsem):
        wid = lax.axis_index("s") * nc + lax.axis_index("c")
        base = pl.multiple_of(lax.div(wid * rows_per_tile, 8) * 8, 8)

        def rd(k, s):
            return pltpu.make_async_copy(
                in_hbm.at[pl.ds(base + offs[k], _CH)], bufs.at[s], rsem.at[s])

        def wr(k, s):
            return pltpu.make_async_copy(
                bufs.at[s], out_hbm.at[pl.ds(base + offs[k], _CH)], wsem.at[s])

        rd(0, 0).start()
        for k in range(n):
            s = k % 2
            rd(k, s).wait()
            if k + 1 < n:
                if k >= 1:
                    wr(k - 1, 1 - s).wait()
                rd(k + 1, 1 - s).start()
            wr(k, s).start()
        if n >= 2:
            wr(n - 2, (n - 2) % 2).wait()
        wr(n - 1, (n - 1) % 2).wait()

    return copy_kernel(item_emb)


# speedup vs baseline: 17.0008x; 1.0087x over previous
import functools

import jax
import jax.numpy as jnp
from jax import lax
from jax.experimental import pallas as pl
from jax.experimental.pallas import tpu as pltpu
from jax.experimental.pallas import tpu_sc as plsc

_N_ROWS = 1_000_000
_CH = 240
_K = 4


def kernel(item_emb):
    info = plsc.get_sparse_core_info()
    nc, ns = info.num_cores, info.num_subcores
    rows_per_tile = _N_ROWS // (nc * ns)
    span = rows_per_tile + 8 - rows_per_tile % 8
    n_full = span // _CH
    offs = [k * _CH for k in range(n_full)]
    if n_full * _CH < span:
        offs.append(span - _CH)
    n = len(offs)

    mesh = plsc.VectorSubcoreMesh(core_axis_name="c", subcore_axis_name="s")

    @functools.partial(
        pl.kernel,
        mesh=mesh,
        out_type=jax.ShapeDtypeStruct(item_emb.shape, item_emb.dtype),
        scratch_types=[
            pltpu.VMEM((_K, _CH, 32), jnp.float32),
            pltpu.SemaphoreType.DMA((_K,)),
            pltpu.SemaphoreType.DMA((_K,)),
        ],
    )
    def copy_kernel(in_hbm, out_hbm, bufs, rsem, wsem):
        wid = lax.axis_index("s") * nc + lax.axis_index("c")
        base = pl.multiple_of(lax.div(wid * rows_per_tile, 8) * 8, 8)

        def rd(k, s):
            return pltpu.make_async_copy(
                in_hbm.at[pl.ds(base + offs[k], _CH)], bufs.at[s], rsem.at[s])

        def wr(k, s):
            return pltpu.make_async_copy(
                bufs.at[s], out_hbm.at[pl.ds(base + offs[k], _CH)], wsem.at[s])

        _L = _K // 2
        for k in range(_L):
            rd(k, k).start()
        for k in range(n):
            s = k % _K
            rd(k, s).wait()
            wr(k, s).start()
            c = k + _L
            if c < n:
                sc = c % _K
                if c >= _K:
                    wr(c - _K, sc).wait()
                rd(c, sc).start()
        for k in range(max(0, n - _K), n):
            wr(k, k % _K).wait()

    return copy_kernel(item_emb)
